# baseline (speedup 1.0000x reference)
import functools

import jax
import jax.numpy as jnp
from jax import lax
from jax.experimental import pallas as pl
from jax.experimental.pallas import tpu as pltpu
from jax.experimental.pallas import tpu_sc as plsc

_VOCAB = 30522
_HID = 768
_S = 128
_B = 1024
_EPS = 1e-12
_L = 16
_NW = 32
_SEQ_PER_W = _B // _NW
_NCHUNK = _S // _L
_JV = _HID // _L
_UNROLL = 8


def _rsqrt16(x):
    i = plsc.bitcast(x, jnp.int32)
    i = jnp.int32(0x5F3759DF) - lax.shift_right_logical(i, 1)
    y = plsc.bitcast(i, jnp.float32)
    for _ in range(4):
        y = y * (jnp.float32(1.5) - jnp.float32(0.5) * x * y * y)
    return y


def _body(ids_hbm, word_hbm, pos_hbm, tt_hbm, g_hbm, b_hbm, out_hbm,
          pos_v, rows_v, idx_v, tt_v, g_v, b_v, gsem):
    cid = lax.axis_index("c")
    sid = lax.axis_index("s")
    wid = sid * 2 + cid

    pltpu.sync_copy(tt_hbm, tt_v)
    pltpu.sync_copy(g_hbm, g_v)
    pltpu.sync_copy(b_hbm, b_v)

    iota = lax.iota(jnp.int32, _L)
    zeros_i = jnp.zeros((_L,), jnp.int32)
    inv_hid = jnp.float32(1.0 / _HID)

    def chunk_body(sci, _):
        pltpu.sync_copy(pos_hbm.at[pl.ds(sci * _L, _L)], pos_v)

        def ttadd(jv, _):
            t = tt_v[pl.ds(jv * _L, _L)]
            for r in range(_L):
                pos_v[r, pl.ds(jv * _L, _L)] = pos_v[r, pl.ds(jv * _L, _L)] + t
            return 0

        lax.fori_loop(0, _JV, ttadd, 0, unroll=False)

        def seq_body(b, _):
            base = (wid * _SEQ_PER_W + b) * _S + sci * _L
            pltpu.sync_copy(ids_hbm.at[pl.ds(base, _L)], idx_v)
            pltpu.async_copy(word_hbm.at[idx_v], rows_v, gsem).wait()

            def p1(jb, carry):
                acc, acc2 = carry
                j0 = jb * _UNROLL
                for u in range(_UNROLL):
                    jj = zeros_i + (j0 + u)
                    x = plsc.load_gather(rows_v, [iota, jj])
                    p = plsc.load_gather(pos_v, [iota, jj])
                    x = x + p
                    plsc.store_scatter(rows_v, [iota, jj], x)
                    acc = acc + x
                    acc2 = acc2 + x * x
                return acc, acc2

            zf = jnp.zeros((_L,), jnp.float32)
            acc, acc2 = lax.fori_loop(0, _HID // _UNROLL, p1, (zf, zf),
                                      unroll=False)
            mean = acc * inv_hid
            var = acc2 * inv_hid - mean * mean
            rs = _rsqrt16(var + jnp.float32(_EPS))

            def p2(jb, _):
                j0 = jb * _L
                gvec = g_v[pl.ds(j0, _L)]
                bvec = b_v[pl.ds(j0, _L)]
                for u in range(_L):
                    jj = zeros_i + (j0 + u)
                    x = plsc.load_gather(rows_v, [iota, jj])
                    y = (x - mean) * (rs * gvec[u]) + bvec[u]
                    plsc.store_scatter(rows_v, [iota, jj], y)
                return 0

            lax.fori_loop(0, _HID // _L, p2, 0, unroll=False)
            pltpu.sync_copy(rows_v, out_hbm.at[pl.ds(base, _L)])
            return 0

        lax.fori_loop(0, _SEQ_PER_W, seq_body, 0, unroll=False)
        return 0

    lax.fori_loop(0, _NCHUNK, chunk_body, 0, unroll=False)


def kernel(input_ids, word_embeddings, position_embeddings,
           token_type_embeddings, ln_gamma, ln_beta):
    ids = input_ids.reshape(-1).astype(jnp.int32)
    pos = position_embeddings[:_S]
    tt = token_type_embeddings[0]

    mesh = plsc.VectorSubcoreMesh(core_axis_name="c", subcore_axis_name="s")
    call = pl.kernel(
        _body,
        out_type=jax.ShapeDtypeStruct((_B * _S, _HID), jnp.float32),
        mesh=mesh,
        compiler_params=pltpu.CompilerParams(needs_layout_passes=False),
        scratch_types=[
            pltpu.VMEM((_L, _HID), jnp.float32),
            pltpu.VMEM((_L, _HID), jnp.float32),
            pltpu.VMEM((_L,), jnp.int32),
            pltpu.VMEM((_HID,), jnp.float32),
            pltpu.VMEM((_HID,), jnp.float32),
            pltpu.VMEM((_HID,), jnp.float32),
            pltpu.SemaphoreType.DMA,
        ],
    )
    out = call(ids, word_embeddings, pos, tt, ln_gamma, ln_beta)
    return out.reshape(_B, _S, _HID)
